# baseline (speedup 1.0000x reference)
import functools

import jax
import jax.numpy as jnp
from jax import lax
from jax.experimental import pallas as pl
from jax.experimental.pallas import tpu as pltpu
from jax.experimental.pallas import tpu_sc as plsc

_NC = 2
_NS = 16
_NW = _NC * _NS

_OUT_DIM = 64
_PAD_DIM = 128
_BPC = 2
_HIST = 50
_CHUNK = _BPC * _HIST
_NBUF = 4


def _make_gather(batch: int):
    n_chunks = batch // _BPC
    assert n_chunks % (_NW * _NBUF) == 0
    chunks_per_w = n_chunks // _NW
    n_groups = chunks_per_w // _NBUF

    mesh = plsc.VectorSubcoreMesh(core_axis_name="c", subcore_axis_name="s")
    scratch = [pltpu.VMEM((chunks_per_w, _CHUNK), jnp.int32)]
    scratch += [pltpu.VMEM((_CHUNK, _PAD_DIM), jnp.float32)] * _NBUF
    scratch += [pltpu.SemaphoreType.DMA] * (2 * _NBUF)

    @functools.partial(
        pl.kernel,
        out_type=jax.ShapeDtypeStruct((batch, _HIST, _OUT_DIM), jnp.float32),
        mesh=mesh,
        scratch_types=scratch,
        compiler_params=pltpu.CompilerParams(use_tc_tiling_on_sc=False),
    )
    def gather(table_hbm, idx_hbm, out_hbm, idx_v, *bufs):
        rows = bufs[:_NBUF]
        gsem = bufs[_NBUF : 2 * _NBUF]
        ssem = bufs[2 * _NBUF :]
        wid = lax.axis_index("s") * _NC + lax.axis_index("c")
        cbase = wid * chunks_per_w
        bbase = cbase * _BPC
        pltpu.sync_copy(idx_hbm.at[pl.ds(cbase, chunks_per_w)], idx_v)

        def gather_copy(j, b):
            return pltpu.make_async_copy(
                table_hbm.at[idx_v.at[j]], rows[b], gsem[b]
            )

        def store_copy(j, b, k):
            return pltpu.make_async_copy(
                rows[b].at[pl.ds(k * _HIST, _HIST), pl.ds(0, _OUT_DIM)],
                out_hbm.at[bbase + j * _BPC + k],
                ssem[b],
            )

        for b in range(_NBUF):
            gather_copy(b, b).start()

        def group(g, carry):
            j0 = g * _NBUF
            for b in range(_NBUF):
                gather_copy(j0 + b, b).wait()
                for k in range(_BPC):
                    store_copy(j0 + b, b, k).start()
            for b in range(_NBUF):
                for k in range(_BPC):
                    store_copy(j0 + b, b, k).wait()
                gather_copy(j0 + _NBUF + b, b).start()
            return carry

        lax.fori_loop(0, n_groups - 1, group, 0)

        j0 = (n_groups - 1) * _NBUF
        for b in range(_NBUF):
            gather_copy(j0 + b, b).wait()
            for k in range(_BPC):
                store_copy(j0 + b, b, k).start()
        for b in range(_NBUF):
            for k in range(_BPC):
                store_copy(j0 + b, b, k).wait()

    return gather


def kernel(input_ids, table):
    batch, hist = input_ids.shape
    table_pad = jnp.pad(table, ((0, 0), (0, _PAD_DIM - _OUT_DIM)))
    idx2d = input_ids.reshape(batch * hist // _CHUNK, _CHUNK).astype(jnp.int32)
    return _make_gather(batch)(table_pad, idx2d)
